# baseline (speedup 1.0000x reference)
import functools

import jax
import jax.numpy as jnp
from jax import lax
from jax.experimental import pallas as pl
from jax.experimental.pallas import tpu as pltpu
from jax.experimental.pallas import tpu_sc as plsc

EPS_ = 1e-12
NT_ = 1000
NSPLIT_ = 8
NSKILL_ = 64
D_ = NSPLIT_ * NSKILL_
B_ = 16384

NC_ = 2
NS_ = 16
NW_ = NC_ * NS_
BPW_ = B_ // NW_
CH_ = 64
NCH_ = BPW_ // CH_


def _norm_body(x_ref, o_ref):
    x = x_ref[:]
    p = 1.0 / (1.0 + jnp.exp(-x))
    j = lax.broadcasted_iota(jnp.int32, (D_, NSPLIT_), 0) // NSKILL_
    k = lax.broadcasted_iota(jnp.int32, (D_, NSPLIT_), 1)
    s_mat = (j == k).astype(jnp.float32)
    denom = jnp.dot(p, s_mat, preferred_element_type=jnp.float32)
    jj = lax.broadcasted_iota(jnp.int32, (NSPLIT_, D_), 1) // NSKILL_
    kk = lax.broadcasted_iota(jnp.int32, (NSPLIT_, D_), 0)
    e_mat = (jj == kk).astype(jnp.float32)
    dnb = jnp.dot(denom, e_mat, preferred_element_type=jnp.float32)
    o_ref[:] = p / (dnb + EPS_)


def _normalize_table(module_logits):
    return pl.pallas_call(
        _norm_body,
        out_shape=jax.ShapeDtypeStruct((NT_, D_), jnp.float32),
    )(module_logits)


NBUF_ = 3


def _gather_body(table_hbm, idx_hbm, out_hbm, idx_v,
                 r0, r1, r2, g0, g1, g2, o0, o1, o2):
    bufs = (r0, r1, r2)
    gsem = (g0, g1, g2)
    osem = (o0, o1, o2)
    wid = lax.axis_index("s") * NC_ + lax.axis_index("c")
    base = wid * BPW_
    pltpu.sync_copy(idx_hbm.at[pl.ds(base, BPW_)], idx_v)

    gh = [None] * NCH_
    oh = [None] * NCH_
    for c in range(NBUF_):
        gh[c] = pltpu.async_copy(
            table_hbm.at[idx_v.at[pl.ds(c * CH_, CH_)]], bufs[c], gsem[c]
        )
    for c in range(NCH_):
        b = c % NBUF_
        gh[c].wait()
        oh[c] = pltpu.async_copy(
            bufs[b], out_hbm.at[pl.ds(base + c * CH_, CH_)], osem[b]
        )
        nxt = c + NBUF_
        if nxt < NCH_:
            oh[c].wait()
            gh[nxt] = pltpu.async_copy(
                table_hbm.at[idx_v.at[pl.ds(nxt * CH_, CH_)]], bufs[b], gsem[b]
            )
    for c in range(max(0, NCH_ - NBUF_), NCH_):
        oh[c].wait()


_sc_gather = functools.partial(
    pl.kernel,
    mesh=plsc.VectorSubcoreMesh(core_axis_name="c", subcore_axis_name="s"),
    out_type=jax.ShapeDtypeStruct((B_, D_), jnp.float32),
    scratch_types=[
        pltpu.VMEM((BPW_,), jnp.int32),
        pltpu.VMEM((CH_, D_), jnp.float32),
        pltpu.VMEM((CH_, D_), jnp.float32),
        pltpu.VMEM((CH_, D_), jnp.float32),
        pltpu.SemaphoreType.DMA,
        pltpu.SemaphoreType.DMA,
        pltpu.SemaphoreType.DMA,
        pltpu.SemaphoreType.DMA,
        pltpu.SemaphoreType.DMA,
        pltpu.SemaphoreType.DMA,
    ],
)(_gather_body)


RB_ = 512


def _relayout_body(x_ref, o_ref):
    for s in range(NSPLIT_):
        o_ref[:, s, :] = x_ref[:, s * NSKILL_:(s + 1) * NSKILL_]


def _relayout(g):
    return pl.pallas_call(
        _relayout_body,
        grid=(B_ // RB_,),
        in_specs=[pl.BlockSpec((RB_, D_), lambda i: (i, 0))],
        out_specs=pl.BlockSpec((RB_, NSPLIT_, NSKILL_), lambda i: (i, 0, 0)),
        out_shape=jax.ShapeDtypeStruct((B_, NSPLIT_, NSKILL_), jnp.float32),
    )(g)


def kernel(task_ids, input_ids, module_logits):
    del input_ids
    table = _normalize_table(module_logits)
    out = _sc_gather(table, task_ids.astype(jnp.int32))
    return _relayout(out)


# speedup vs baseline: 2.3915x; 2.3915x over previous
import functools

import jax
import jax.numpy as jnp
from jax import lax
from jax.experimental import pallas as pl
from jax.experimental.pallas import tpu as pltpu
from jax.experimental.pallas import tpu_sc as plsc

EPS_ = 1e-12
NT_ = 1000
NSPLIT_ = 8
NSKILL_ = 64
D_ = NSPLIT_ * NSKILL_
B_ = 16384

NC_ = 2
NS_ = 16
NW_ = NC_ * NS_
BPW_ = B_ // NW_
CH_ = 32
NCH_ = BPW_ // CH_


def _norm_body(x_ref, o_ref):
    x = x_ref[:]
    p = 1.0 / (1.0 + jnp.exp(-x))
    j = lax.broadcasted_iota(jnp.int32, (D_, NSPLIT_), 0) // NSKILL_
    k = lax.broadcasted_iota(jnp.int32, (D_, NSPLIT_), 1)
    s_mat = (j == k).astype(jnp.float32)
    denom = jnp.dot(p, s_mat, preferred_element_type=jnp.float32)
    jj = lax.broadcasted_iota(jnp.int32, (NSPLIT_, D_), 1) // NSKILL_
    kk = lax.broadcasted_iota(jnp.int32, (NSPLIT_, D_), 0)
    e_mat = (jj == kk).astype(jnp.float32)
    dnb = jnp.dot(denom, e_mat, preferred_element_type=jnp.float32)
    o_ref[:] = p / (dnb + EPS_)


def _normalize_table(module_logits):
    return pl.pallas_call(
        _norm_body,
        out_shape=jax.ShapeDtypeStruct((NT_, D_), jnp.float32),
    )(module_logits)


NBUF_ = 6


def _gather_body(table_hbm, idx_hbm, out_hbm, idx_v,
                 r0, r1, r2, r3, r4, r5, g0, g1, g2, g3, g4, g5,
                 o0, o1, o2, o3, o4, o5):
    bufs = (r0, r1, r2, r3, r4, r5)
    gsem = (g0, g1, g2, g3, g4, g5)
    osem = (o0, o1, o2, o3, o4, o5)
    wid = lax.axis_index("s") * NC_ + lax.axis_index("c")
    base = wid * BPW_
    pltpu.sync_copy(idx_hbm.at[pl.ds(base, BPW_)], idx_v)

    gh = [None] * NCH_
    oh = [None] * NCH_
    for c in range(NBUF_):
        gh[c] = pltpu.async_copy(
            table_hbm.at[idx_v.at[pl.ds(c * CH_, CH_)]], bufs[c], gsem[c]
        )
    for c in range(NCH_):
        b = c % NBUF_
        gh[c].wait()
        oh[c] = pltpu.async_copy(
            bufs[b], out_hbm.at[pl.ds(base + c * CH_, CH_)], osem[b]
        )
        nxt = c + NBUF_
        if nxt < NCH_:
            oh[c].wait()
            gh[nxt] = pltpu.async_copy(
                table_hbm.at[idx_v.at[pl.ds(nxt * CH_, CH_)]], bufs[b], gsem[b]
            )
    for c in range(max(0, NCH_ - NBUF_), NCH_):
        oh[c].wait()


_sc_gather = functools.partial(
    pl.kernel,
    mesh=plsc.VectorSubcoreMesh(core_axis_name="c", subcore_axis_name="s"),
    out_type=jax.ShapeDtypeStruct((B_, D_), jnp.float32),
    scratch_types=[
        pltpu.VMEM((BPW_,), jnp.int32),
        pltpu.VMEM((CH_, D_), jnp.float32),
        pltpu.VMEM((CH_, D_), jnp.float32),
        pltpu.VMEM((CH_, D_), jnp.float32),
        pltpu.VMEM((CH_, D_), jnp.float32),
        pltpu.VMEM((CH_, D_), jnp.float32),
        pltpu.VMEM((CH_, D_), jnp.float32),
        pltpu.SemaphoreType.DMA,
        pltpu.SemaphoreType.DMA,
        pltpu.SemaphoreType.DMA,
        pltpu.SemaphoreType.DMA,
        pltpu.SemaphoreType.DMA,
        pltpu.SemaphoreType.DMA,
        pltpu.SemaphoreType.DMA,
        pltpu.SemaphoreType.DMA,
        pltpu.SemaphoreType.DMA,
        pltpu.SemaphoreType.DMA,
        pltpu.SemaphoreType.DMA,
        pltpu.SemaphoreType.DMA,
    ],
)(_gather_body)


def kernel(task_ids, input_ids, module_logits):
    del input_ids
    table = _normalize_table(module_logits)
    out = _sc_gather(table, task_ids.astype(jnp.int32))
    return out.reshape(B_, NSPLIT_, NSKILL_)
